# baseline (speedup 1.0000x reference)
import functools

import jax
import jax.numpy as jnp
from jax import lax
from jax.experimental import pallas as pl
from jax.experimental.pallas import tpu as pltpu
from jax.experimental.pallas import tpu_sc as plsc

S, B, D = 250, 128, 1024
L = 16
NC, NS = 2, 16
NW = NC * NS
SP = NW * 8
R_CHUNK = 32
H = B // R_CHUNK
CVG = 8

_mesh = plsc.VectorSubcoreMesh(
    core_axis_name="c", subcore_axis_name="s", num_cores=NC, num_subcores=NS
)


@functools.partial(
    pl.kernel,
    out_type=jax.ShapeDtypeStruct((S * B, D), jnp.float32),
    mesh=_mesh,
    scratch_types=[
        pltpu.VMEM((R_CHUNK, D), jnp.float32),
        pltpu.VMEM((R_CHUNK, D), jnp.float32),
        pltpu.VMEM((R_CHUNK, D), jnp.float32),
        pltpu.VMEM((8, D), jnp.float32),
        pltpu.SemaphoreType.DMA,
        pltpu.SemaphoreType.DMA,
        pltpu.SemaphoreType.DMA,
        pltpu.SemaphoreType.DMA,
        pltpu.SemaphoreType.DMA,
        pltpu.SemaphoreType.DMA,
    ],
    compiler_params=pltpu.CompilerParams(
        use_tc_tiling_on_sc=True,
        disable_bounds_checks=True,
        disable_semaphore_checks=True,
    ),
)
def _sc_pe_add(
    x_hbm, pe_hbm, out_hbm, xb0, xb1, xb2, pebuf, si0, si1, si2, so0, so1, so2
):
    wid = lax.axis_index("s") * NC + lax.axis_index("c")
    s_base = wid * 8
    n = jnp.minimum(8, S - s_base) * H
    xbufs = (xb0, xb1, xb2)
    in_sems = (si0, si1, si2)
    out_sems = (so0, so1, so2)

    def row0_of(c):
        return (s_base + c // H) * B + (c % H) * R_CHUNK

    def in_copy(c, k):
        return pltpu.make_async_copy(
            x_hbm.at[pl.ds(row0_of(c), R_CHUNK), :], xbufs[k], in_sems[k]
        )

    def in_start(c, k):
        hr = R_CHUNK // 2
        r0 = row0_of(c)
        pltpu.make_async_copy(
            x_hbm.at[pl.ds(r0, hr), :],
            xbufs[k].at[pl.ds(0, hr), :],
            in_sems[k],
        ).start()
        pltpu.make_async_copy(
            x_hbm.at[pl.ds(r0 + hr, hr), :],
            xbufs[k].at[pl.ds(hr, hr), :],
            in_sems[k],
        ).start()

    def out_copy(c, k):
        return pltpu.make_async_copy(
            xbufs[k], out_hbm.at[pl.ds(row0_of(c), R_CHUNK), :], out_sems[k]
        )

    def out_start(c, k):
        hr = R_CHUNK // 2
        r0 = row0_of(c)
        pltpu.make_async_copy(
            xbufs[k].at[pl.ds(0, hr), :],
            out_hbm.at[pl.ds(r0, hr), :],
            out_sems[k],
        ).start()
        pltpu.make_async_copy(
            xbufs[k].at[pl.ds(hr, hr), :],
            out_hbm.at[pl.ds(r0 + hr, hr), :],
            out_sems[k],
        ).start()

    in_start(0, 0)
    in_start(1, 1)
    pltpu.sync_copy(pe_hbm.at[pl.ds(s_base, 8), :], pebuf)

    def phase(c, k):
        in_copy(c, k).wait()
        k2 = (k + 2) % 3

        @pl.when(c + 2 < n)
        def _prefetch():
            @pl.when(c > 0)
            def _drain():
                out_copy(c - 1, k2).wait()

            in_start(c + 2, k2)

        s_off = c // H
        buf = xbufs[k]
        for g in range(D // (L * CVG)):
            pevs = tuple(
                pebuf[s_off, pl.ds((g * CVG + v) * L, L)] for v in range(CVG)
            )

            @plsc.parallel_loop(0, R_CHUNK, unroll=2)
            def _row(r, _pevs=pevs, _g=g):
                for v, pv in enumerate(_pevs):
                    col = (_g * CVG + v) * L
                    buf[r, pl.ds(col, L)] = buf[r, pl.ds(col, L)] + pv

        out_start(c, k)

    @pl.loop(0, (n + 2) // 3)
    def _ring(g3):
        for p in range(3):
            c = g3 * 3 + p

            @pl.when(c < n)
            def _run(c=c, p=p):
                phase(c, p)

    out_copy(n - 3, 2).wait()
    out_copy(n - 2, 0).wait()
    out_copy(n - 1, 1).wait()


def kernel(x, pos_embed):
    pe_pad = jnp.zeros((SP, D), jnp.float32).at[:S].set(pos_embed[:S])
    out = _sc_pe_add(x.reshape(S * B, D), pe_pad)
    return out.reshape(S, B, D)


# speedup vs baseline: 1.0050x; 1.0050x over previous
import functools

import jax
import jax.numpy as jnp
from jax import lax
from jax.experimental import pallas as pl
from jax.experimental.pallas import tpu as pltpu
from jax.experimental.pallas import tpu_sc as plsc

S, B, D = 250, 128, 1024
L = 16
NC, NS = 2, 16
NW = NC * NS
SP = NW * 8
R_CHUNK = 32
H = B // R_CHUNK
CVG = 8

_mesh = plsc.VectorSubcoreMesh(
    core_axis_name="c", subcore_axis_name="s", num_cores=NC, num_subcores=NS
)


@functools.partial(
    pl.kernel,
    out_type=jax.ShapeDtypeStruct((S * B, D), jnp.float32),
    mesh=_mesh,
    scratch_types=[
        pltpu.VMEM((R_CHUNK, D), jnp.float32),
        pltpu.VMEM((R_CHUNK, D), jnp.float32),
        pltpu.VMEM((R_CHUNK, D), jnp.float32),
        pltpu.VMEM((8, D), jnp.float32),
        pltpu.SemaphoreType.DMA,
        pltpu.SemaphoreType.DMA,
        pltpu.SemaphoreType.DMA,
        pltpu.SemaphoreType.DMA,
        pltpu.SemaphoreType.DMA,
        pltpu.SemaphoreType.DMA,
    ],
    compiler_params=pltpu.CompilerParams(
        use_tc_tiling_on_sc=True,
        disable_bounds_checks=True,
        disable_semaphore_checks=True,
    ),
)
def _sc_pe_add(
    x_hbm, pe_hbm, out_hbm, xb0, xb1, xb2, pebuf, si0, si1, si2, so0, so1, so2
):
    wid = lax.axis_index("s") * NC + lax.axis_index("c")
    s_base = wid * 8
    n = jnp.minimum(8, S - s_base) * H
    xbufs = (xb0, xb1, xb2)
    in_sems = (si0, si1, si2)
    out_sems = (so0, so1, so2)

    def row0_of(c):
        return (s_base + c // H) * B + (c % H) * R_CHUNK

    def in_copy(c, k):
        return pltpu.make_async_copy(
            x_hbm.at[pl.ds(row0_of(c), R_CHUNK), :], xbufs[k], in_sems[k]
        )

    def out_copy(c, k):
        return pltpu.make_async_copy(
            xbufs[k], out_hbm.at[pl.ds(row0_of(c), R_CHUNK), :], out_sems[k]
        )

    in_copy(0, 0).start()
    in_copy(1, 1).start()
    pltpu.sync_copy(pe_hbm.at[pl.ds(s_base, 8), :], pebuf)

    def phase(c, k):
        in_copy(c, k).wait()
        k2 = (k + 2) % 3

        @pl.when(c + 2 < n)
        def _prefetch():
            @pl.when(c > 0)
            def _drain():
                out_copy(c - 1, k2).wait()

            in_copy(c + 2, k2).start()

        s_off = c // H
        buf = xbufs[k]
        for g in range(D // (L * CVG)):
            pevs = tuple(
                pebuf[s_off, pl.ds((g * CVG + v) * L, L)] for v in range(CVG)
            )

            @plsc.parallel_loop(0, R_CHUNK, unroll=2)
            def _row(r, _pevs=pevs, _g=g):
                for v, pv in enumerate(_pevs):
                    col = (_g * CVG + v) * L
                    buf[r, pl.ds(col, L)] = buf[r, pl.ds(col, L)] + pv

        out_copy(c, k).start()

    @pl.loop(0, (n + 2) // 3)
    def _ring(g3):
        for p in range(3):
            c = g3 * 3 + p

            @pl.when(c < n)
            def _run(c=c, p=p):
                phase(c, p)

    out_copy(n - 3, 2).wait()
    out_copy(n - 2, 0).wait()
    out_copy(n - 1, 1).wait()


def kernel(x, pos_embed):
    pe_pad = jnp.zeros((SP, D), jnp.float32).at[:S].set(pos_embed[:S])
    out = _sc_pe_add(x.reshape(S * B, D), pe_pad)
    return out.reshape(S, B, D)
